# baseline (speedup 1.0000x reference)
import functools

import jax
import jax.numpy as jnp
from jax import lax
from jax.experimental import pallas as pl
from jax.experimental.pallas import tpu as pltpu
from jax.experimental.pallas import tpu_sc as plsc

B_TOK = 4096
SEQ = 200
D = 64
B = B_TOK * SEQ
NC, NS = 2, 16
NW = NC * NS
B_PER_W = B // NW
CHUNK = 128
N_CHUNKS = B_PER_W // CHUNK


def _make_sc_gather():
    mesh = plsc.VectorSubcoreMesh(core_axis_name="c", subcore_axis_name="s")

    @functools.partial(
        pl.kernel,
        mesh=mesh,
        out_type=jax.ShapeDtypeStruct((B, D), jnp.float32),
        scratch_types=[
            pltpu.VMEM((N_CHUNKS, CHUNK), jnp.int32),
            pltpu.VMEM((CHUNK, D), jnp.float32),
            pltpu.SemaphoreType.DMA,
        ],
        compiler_params=pltpu.CompilerParams(use_tc_tiling_on_sc=False),
    )
    def gather_kernel(idx_hbm, table_hbm, out_hbm, idx_v, rows_v, sem):
        wid = lax.axis_index("s") * NC + lax.axis_index("c")
        base = wid * B_PER_W
        pltpu.sync_copy(idx_hbm.at[wid], idx_v)

        def body(j, carry):
            pltpu.async_copy(table_hbm.at[idx_v.at[j]], rows_v, sem).wait()
            pltpu.sync_copy(rows_v, out_hbm.at[pl.ds(base + j * CHUNK, CHUNK)])
            return carry

        lax.fori_loop(0, N_CHUNKS, body, 0)

    return gather_kernel


_sc_gather = _make_sc_gather()


@jax.jit
def kernel(token_ids, embedding):
    idx = token_ids.reshape(NW, N_CHUNKS, CHUNK).astype(jnp.int32)
    out = _sc_gather(idx, embedding)
    return out.reshape(B_TOK, SEQ, D)


# speedup vs baseline: 1.1113x; 1.1113x over previous
import functools

import jax
import jax.numpy as jnp
from jax import lax
from jax.experimental import pallas as pl
from jax.experimental.pallas import tpu as pltpu
from jax.experimental.pallas import tpu_sc as plsc

B_TOK = 4096
SEQ = 200
D = 64
B = B_TOK * SEQ
NC, NS = 2, 16
NW = NC * NS
B_PER_W = B // NW
CHUNK = 128
N_CHUNKS = B_PER_W // CHUNK
NBUF = 4
N_GROUPS = N_CHUNKS // NBUF


def _make_sc_gather():
    mesh = plsc.VectorSubcoreMesh(core_axis_name="c", subcore_axis_name="s")

    @functools.partial(
        pl.kernel,
        mesh=mesh,
        out_type=jax.ShapeDtypeStruct((B, D), jnp.float32),
        scratch_types=[
            pltpu.VMEM((N_CHUNKS, CHUNK), jnp.int32),
            pltpu.VMEM((NBUF, CHUNK, D), jnp.float32),
            pltpu.SemaphoreType.DMA((NBUF,)),
            pltpu.SemaphoreType.DMA((NBUF,)),
        ],
        compiler_params=pltpu.CompilerParams(use_tc_tiling_on_sc=False),
    )
    def gather_kernel(idx_hbm, table_hbm, out_hbm, idx_v, rows_v, gsem, ssem):
        wid = lax.axis_index("s") * NC + lax.axis_index("c")
        base = wid * B_PER_W
        pltpu.sync_copy(idx_hbm.at[wid], idx_v)

        def start_gather(j, b):
            pltpu.async_copy(table_hbm.at[idx_v.at[j]], rows_v.at[b],
                             gsem.at[b])

        def wait_gather(j, b):
            pltpu.make_async_copy(table_hbm.at[idx_v.at[j]], rows_v.at[b],
                                  gsem.at[b]).wait()

        def start_store(j, b):
            pltpu.async_copy(rows_v.at[b],
                             out_hbm.at[pl.ds(base + j * CHUNK, CHUNK)],
                             ssem.at[b])

        def wait_store(j, b):
            pltpu.make_async_copy(rows_v.at[b],
                                  out_hbm.at[pl.ds(base + j * CHUNK, CHUNK)],
                                  ssem.at[b]).wait()

        for b in range(NBUF):
            start_gather(b, b)

        def outer(i, carry):
            g = i * NBUF
            for b in range(NBUF):
                wait_gather(g + b, b)
                start_store(g + b, b)
            for b in range(NBUF):
                wait_store(g + b, b)
                start_gather(g + NBUF + b, b)
            return carry

        lax.fori_loop(0, N_GROUPS - 1, outer, 0)

        g = (N_GROUPS - 1) * NBUF
        for b in range(NBUF):
            wait_gather(g + b, b)
            start_store(g + b, b)
        for b in range(NBUF):
            wait_store(g + b, b)

    return gather_kernel


_sc_gather = _make_sc_gather()


@jax.jit
def kernel(token_ids, embedding):
    idx = token_ids.reshape(NW, N_CHUNKS, CHUNK).astype(jnp.int32)
    out = _sc_gather(idx, embedding)
    return out.reshape(B_TOK, SEQ, D)


# speedup vs baseline: 1.1141x; 1.0026x over previous
import functools

import jax
import jax.numpy as jnp
from jax import lax
from jax.experimental import pallas as pl
from jax.experimental.pallas import tpu as pltpu
from jax.experimental.pallas import tpu_sc as plsc

B_TOK = 4096
SEQ = 200
D = 64
B = B_TOK * SEQ
NC, NS = 2, 16
NW = NC * NS
B_PER_W = B // NW
CHUNK = 128
N_STREAMS = B_PER_W // CHUNK
NBUF = 8
N_GROUPS = N_STREAMS // NBUF


def _make_sc_gather():
    mesh = plsc.VectorSubcoreMesh(core_axis_name="c", subcore_axis_name="s")

    @functools.partial(
        pl.kernel,
        mesh=mesh,
        out_type=jax.ShapeDtypeStruct((NW, N_STREAMS, CHUNK, D), jnp.float32),
        scratch_types=[
            pltpu.VMEM((N_STREAMS, CHUNK), jnp.int32),
            pltpu.VMEM((NBUF, CHUNK, D), jnp.float32),
            pltpu.SemaphoreType.DMA((NBUF,)),
            pltpu.SemaphoreType.DMA((NBUF,)),
        ],
        compiler_params=pltpu.CompilerParams(use_tc_tiling_on_sc=False),
    )
    def gather_kernel(idx_hbm, table_hbm, out_hbm, idx_v, rows_v, gsem, ssem):
        wid = lax.axis_index("s") * NC + lax.axis_index("c")
        pltpu.sync_copy(idx_hbm.at[wid], idx_v)

        def start_gather(j, b):
            pltpu.async_copy(table_hbm.at[idx_v.at[j]], rows_v.at[b],
                             gsem.at[b])

        def wait_gather(j, b):
            pltpu.make_async_copy(table_hbm.at[idx_v.at[j]], rows_v.at[b],
                                  gsem.at[b]).wait()

        def start_store(j, b):
            pltpu.async_copy(rows_v.at[b], out_hbm.at[wid].at[j], ssem.at[b])

        def wait_store(j, b):
            pltpu.make_async_copy(rows_v.at[b], out_hbm.at[wid].at[j],
                                  ssem.at[b]).wait()

        for b in range(NBUF):
            start_gather(b, b)

        def outer(i, carry):
            g = i * NBUF
            for b in range(NBUF):
                wait_gather(g + b, b)
                start_store(g + b, b)
            for b in range(NBUF):
                wait_store(g + b, b)
                start_gather(g + NBUF + b, b)
            return carry

        lax.fori_loop(0, N_GROUPS - 1, outer, 0)

        g = (N_GROUPS - 1) * NBUF
        for b in range(NBUF):
            wait_gather(g + b, b)
            start_store(g + b, b)
        for b in range(NBUF):
            wait_store(g + b, b)

    return gather_kernel


_sc_gather = _make_sc_gather()


@jax.jit
def kernel(token_ids, embedding):
    idx = token_ids.reshape(NW, N_STREAMS, CHUNK).astype(jnp.int32)
    out = _sc_gather(idx, embedding)
    return out.reshape(B_TOK, SEQ, D)
